# baseline (speedup 1.0000x reference)
import functools

import jax
import jax.numpy as jnp
from jax import lax
from jax.experimental import pallas as pl
from jax.experimental.pallas import tpu as pltpu
from jax.experimental.pallas import tpu_sc as plsc

_INTERPRET = False
_USE_SC = True

C = 768
M = 8
K = 8
CV = C // M
H = 32
W = 32
BS = 4
HW = H * W
N = BS * HW
T = BS * M
NPOS = K * HW

BLK = 512
GRID_MM = N // BLK

CHUNKP = 256
NCHUNKP = NPOS // CHUNKP

_SIGMA = [32 * (p // 32) + (p % 2) * 16 + (p % 32) // 2 for p in range(CV)]


def _stage_a_body(q_ref, f_ref, wz_ref, bz_ref, woff_ref, boff_ref,
                  watt_ref, batt_ref, wp_ref, bp_ref,
                  off_ref, att_ref, wpo_ref):
    zq = jnp.dot(q_ref[...], wz_ref[...],
                 preferred_element_type=jnp.float32) + bz_ref[...]
    off_ref[...] = jnp.dot(zq, woff_ref[...],
                           preferred_element_type=jnp.float32) + boff_ref[...]
    att_ref[...] = jnp.dot(zq, watt_ref[...],
                           preferred_element_type=jnp.float32) + batt_ref[...]
    wpo_ref[...] = jnp.dot(f_ref[...], wp_ref[...],
                           preferred_element_type=jnp.float32) + bp_ref[...]


def _stage_a(q2, f2, wzT, bz2, woffT, boff2, wattT, batt2, wpT, bp2):
    row = lambda i: (i, 0)
    rep = lambda i: (0, 0)
    return pl.pallas_call(
        _stage_a_body,
        grid=(GRID_MM,),
        in_specs=[
            pl.BlockSpec((BLK, C), row),
            pl.BlockSpec((BLK, C), row),
            pl.BlockSpec((C, C), rep),
            pl.BlockSpec((1, C), rep),
            pl.BlockSpec((C, 2 * M * K), rep),
            pl.BlockSpec((1, 2 * M * K), rep),
            pl.BlockSpec((C, M * K), rep),
            pl.BlockSpec((1, M * K), rep),
            pl.BlockSpec((C, C), rep),
            pl.BlockSpec((1, C), rep),
        ],
        out_specs=[
            pl.BlockSpec((BLK, 2 * M * K), row),
            pl.BlockSpec((BLK, M * K), row),
            pl.BlockSpec((BLK, C), row),
        ],
        out_shape=[
            jax.ShapeDtypeStruct((N, 2 * M * K), jnp.float32),
            jax.ShapeDtypeStruct((N, M * K), jnp.float32),
            jax.ShapeDtypeStruct((N, C), jnp.float32),
        ],
        interpret=_INTERPRET,
    )(q2, f2, wzT, bz2, woffT, boff2, wattT, batt2, wpT, bp2)


def _stage_b_body(offx_ref, offy_ref, phix_ref, phiy_ref,
                  i00, i10, i01, i11, c00, c10, c01, c11):
    ix = (phix_ref[...] + offx_ref[...]) * (W / (W - 1.0)) - 0.5
    iy = (phiy_ref[...] + offy_ref[...]) * (H / (H - 1.0)) - 0.5
    x0 = jnp.floor(ix)
    y0 = jnp.floor(iy)
    wx1 = ix - x0
    wy1 = iy - y0
    wx0 = 1.0 - wx1
    wy0 = 1.0 - wy1
    x1 = x0 + 1.0
    y1 = y0 + 1.0

    def tap(xf, yf, wgt, iref, cref):
        valid = ((xf >= 0) & (xf <= W - 1) & (yf >= 0) & (yf <= H - 1))
        xc = jnp.clip(xf, 0.0, W - 1.0)
        yc = jnp.clip(yf, 0.0, H - 1.0)
        iref[...] = (yc * W + xc).astype(jnp.int32)
        cref[...] = wgt * valid.astype(jnp.float32)

    tap(x0, y0, wx0 * wy0, i00, c00)
    tap(x1, y0, wx1 * wy0, i10, c10)
    tap(x0, y1, wx0 * wy1, i01, c01)
    tap(x1, y1, wx1 * wy1, i11, c11)


def _stage_b(offx, offy, phix, phiy):
    TB = 4
    blk = lambda i: (i, 0, 0)
    out_spec = pl.BlockSpec((TB, K, HW), blk)
    return pl.pallas_call(
        _stage_b_body,
        grid=(T // TB,),
        in_specs=[
            pl.BlockSpec((TB, K, HW), blk),
            pl.BlockSpec((TB, K, HW), blk),
            pl.BlockSpec((TB, 1, HW), blk),
            pl.BlockSpec((TB, 1, HW), blk),
        ],
        out_specs=[out_spec] * 8,
        out_shape=[jax.ShapeDtypeStruct((T, K, HW), jnp.int32)] * 4
        + [jax.ShapeDtypeStruct((T, K, HW), jnp.float32)] * 4,
        interpret=_INTERPRET,
    )(offx, offy, phix, phiy)


def _sc_sample(tables, idxs, cfs):
    mesh = plsc.VectorSubcoreMesh(core_axis_name="c", subcore_axis_name="s")

    @functools.partial(
        pl.kernel,
        out_type=jax.ShapeDtypeStruct((T, NPOS * CV // 2), jnp.int32),
        mesh=mesh,
        scratch_types=[
            pltpu.VMEM((HW * CV,), jnp.float32),
            [pltpu.VMEM((CHUNKP,), jnp.int32) for _ in range(4)],
            [pltpu.VMEM((CHUNKP,), jnp.float32) for _ in range(4)],
            pltpu.VMEM((CHUNKP * CV // 2,), jnp.int32),
        ],
    )
    def samp(tab_hbm, i0, i1, i2, i3, c0, c1, c2, c3, out_hbm,
             tab_v, idx_vs, cf_vs, out_v):
        t = lax.axis_index("s") * 2 + lax.axis_index("c")
        pltpu.sync_copy(tab_hbm.at[t], tab_v)
        idx_hs = [i0, i1, i2, i3]
        cf_hs = [c0, c1, c2, c3]

        def to_bf_bits(acc):
            xi = lax.bitcast_convert_type(acc, jnp.int32)
            return xi + 0x7FFF + ((xi >> 16) & 1)

        def chunk_body(ci, carry):
            base = ci * CHUNKP
            for j in range(4):
                pltpu.sync_copy(idx_hs[j].at[t, pl.ds(base, CHUNKP)], idx_vs[j])
                pltpu.sync_copy(cf_hs[j].at[t, pl.ds(base, CHUNKP)], cf_vs[j])

            def g_body(g, carry2):
                ivs = [idx_vs[j][pl.ds(g * 16, 16)] for j in range(4)]
                wvs = [cf_vs[j][pl.ds(g * 16, 16)] for j in range(4)]
                for p in range(16):
                    accs = [jnp.zeros((16,), jnp.float32)
                            for _ in range(CV // 16)]
                    for j in range(4):
                        lin = ivs[j][p]
                        wgt = wvs[j][p]
                        for c in range(CV // 16):
                            accs[c] = accs[c] + wgt * tab_v[
                                pl.ds(lin * CV + c * 16, 16)]
                    for b in range(CV // 32):
                        lo = to_bf_bits(accs[2 * b])
                        hi = to_bf_bits(accs[2 * b + 1])
                        word = ((lo >> 16) & 0xFFFF) | (
                            hi & jnp.int32(-65536))
                        out_v[pl.ds((g * 16 + p) * (CV // 2) + b * 16,
                                    16)] = word
                return carry2

            lax.fori_loop(0, CHUNKP // 16, g_body, 0)
            pltpu.sync_copy(out_v, out_hbm.at[
                t, pl.ds(base * (CV // 2), CHUNKP * CV // 2)])
            return carry

        lax.fori_loop(0, NCHUNKP, chunk_body, 0)

    return samp(tables, idxs[0], idxs[1], idxs[2], idxs[3],
                cfs[0], cfs[1], cfs[2], cfs[3])


def _jnp_sample(tables, idxs, cfs):
    sig = jnp.asarray(_SIGMA, dtype=jnp.int32)
    tab3 = tables.reshape(T, HW, CV)
    idx4 = jnp.stack(idxs, axis=2)
    cf4 = jnp.stack(cfs, axis=2)
    rows = jax.vmap(lambda tab, ii: tab[ii])(tab3, idx4)
    samp = jnp.einsum('tpjc,tpj->tpc', rows, cf4)
    samp_bf = samp[..., sig].astype(jnp.bfloat16).reshape(T, NPOS * CV // 2, 2)
    return lax.bitcast_convert_type(samp_bf, jnp.int32)


def _stage_d_body(s_ref, a_ref, ht_ref, g_ref, o_ref):
    a = a_ref[...]
    amax = jnp.max(a, axis=1, keepdims=True)
    e = jnp.exp(a - amax)
    attn = e / jnp.sum(e, axis=1, keepdims=True)
    ab = jnp.dot(attn, ht_ref[...],
                 preferred_element_type=jnp.float32)
    p = s_ref[...].astype(jnp.float32) * ab
    o_ref[...] = jnp.dot(p, g_ref[...],
                         preferred_element_type=jnp.float32)


def _stage_d(stacked2, attn2, htile, gsum):
    row = lambda i: (i, 0)
    rep = lambda i: (0, 0)
    return pl.pallas_call(
        _stage_d_body,
        grid=(T,),
        in_specs=[
            pl.BlockSpec((HW, C), row),
            pl.BlockSpec((HW, K), row),
            pl.BlockSpec((K, C), rep),
            pl.BlockSpec((C, CV), rep),
        ],
        out_specs=pl.BlockSpec((HW, CV), row),
        out_shape=jax.ShapeDtypeStruct((T * HW, CV), jnp.float32),
        interpret=_INTERPRET,
    )(stacked2, attn2, htile, gsum)


def _stage_e_body(x_ref, w_ref, b_ref, o_ref):
    o_ref[...] = jnp.dot(x_ref[...], w_ref[...],
                         preferred_element_type=jnp.float32) + b_ref[...]


def _stage_e(x2, wmT, bm2):
    row = lambda i: (i, 0)
    rep = lambda i: (0, 0)
    return pl.pallas_call(
        _stage_e_body,
        grid=(GRID_MM,),
        in_specs=[
            pl.BlockSpec((BLK, C), row),
            pl.BlockSpec((C, C), rep),
            pl.BlockSpec((1, C), rep),
        ],
        out_specs=pl.BlockSpec((BLK, C), row),
        out_shape=jax.ShapeDtypeStruct((N, C), jnp.float32),
        interpret=_INTERPRET,
    )(x2, wmT, bm2)


def kernel(q, features, ref, Wz, bz, Woff, boff, Watt, batt, Wp, bp, Wm, bm):
    q2 = q.reshape(N, C)
    f2 = features[0].reshape(N, C)

    off_raw, att_raw, wp2 = _stage_a(
        q2, f2,
        Wz.T, bz.reshape(1, C),
        Woff.T, boff.reshape(1, 2 * M * K),
        Watt.T, batt.reshape(1, M * K),
        Wp.T, bp.reshape(1, C))

    off6 = off_raw.reshape(BS, H, W, M, K, 2)
    offx = jnp.transpose(off6[..., 0], (0, 3, 4, 2, 1)).reshape(T, K, HW)
    offy = jnp.transpose(off6[..., 1], (0, 3, 4, 2, 1)).reshape(T, K, HW)
    phix = jnp.tile(jnp.transpose(ref[..., 0], (0, 2, 1)).reshape(BS, 1, HW)
                    * (W - 1.0), (M, 1, 1))
    phiy = jnp.tile(jnp.transpose(ref[..., 1], (0, 2, 1)).reshape(BS, 1, HW)
                    * (H - 1.0), (M, 1, 1))

    i00, i10, i01, i11, c00, c10, c01, c11 = _stage_b(offx, offy, phix, phiy)

    idxs = [a.reshape(T, NPOS) for a in (i00, i10, i01, i11)]
    cfs = [a.reshape(T, NPOS) for a in (c00, c10, c01, c11)]

    sig = jnp.asarray(_SIGMA, dtype=jnp.int32)
    tables = wp2.reshape(BS, HW, M, CV).transpose(0, 2, 1, 3).reshape(
        T, HW * CV)

    if _USE_SC:
        samp = _sc_sample(tables, idxs, cfs)
    else:
        samp = _jnp_sample(tables, idxs, cfs)

    stacked2 = lax.bitcast_convert_type(samp, jnp.bfloat16).reshape(T * HW, C)
    attn2 = att_raw.reshape(BS, HW, M, K).transpose(0, 2, 1, 3).reshape(T * HW, K)

    ii = jnp.arange(C, dtype=jnp.int32)
    e_sem = (ii // CV) * CV + sig[ii % CV]
    htile = (e_sem[None, :] % K == jnp.arange(K, dtype=jnp.int32)[:, None]
             ).astype(jnp.float32)
    gsum = (e_sem[:, None] // K == jnp.arange(CV, dtype=jnp.int32)[None, :]
            ).astype(jnp.float32)

    att_out2 = _stage_d(stacked2, attn2, htile, gsum)

    att_out = att_out2.reshape(BS, M, HW, CV).transpose(0, 2, 1, 3).reshape(N, C)
    final = _stage_e(att_out, Wm.T, bm.reshape(1, C))
    return final.reshape(BS, H, W, C)


# speedup vs baseline: 18.0992x; 18.0992x over previous
import functools

import jax
import jax.numpy as jnp
from jax import lax
from jax.experimental import pallas as pl
from jax.experimental.pallas import tpu as pltpu
from jax.experimental.pallas import tpu_sc as plsc

_INTERPRET = False
_USE_SC = True

C = 768
M = 8
K = 8
CV = C // M
H = 32
W = 32
BS = 4
HW = H * W
N = BS * HW
T = BS * M
NPOS = K * HW

BLK = 512
GRID_MM = N // BLK

CHUNKP = 256
NCHUNKP = NPOS // CHUNKP

_SIGMA = [32 * (p // 32) + (p % 2) * 16 + (p % 32) // 2 for p in range(CV)]


def _stage_a_body(q_ref, f_ref, wz_ref, bz_ref, woff_ref, boff_ref,
                  watt_ref, batt_ref, wp_ref, bp_ref,
                  off_ref, att_ref, wpo_ref):
    zq = jnp.dot(q_ref[...], wz_ref[...],
                 preferred_element_type=jnp.float32) + bz_ref[...]
    off_ref[...] = jnp.dot(zq, woff_ref[...],
                           preferred_element_type=jnp.float32) + boff_ref[...]
    att_ref[...] = jnp.dot(zq, watt_ref[...],
                           preferred_element_type=jnp.float32) + batt_ref[...]
    wpo_ref[...] = jnp.dot(f_ref[...], wp_ref[...],
                           preferred_element_type=jnp.float32) + bp_ref[...]


def _stage_a(q2, f2, wzT, bz2, woffT, boff2, wattT, batt2, wpT, bp2):
    row = lambda i: (i, 0)
    rep = lambda i: (0, 0)
    return pl.pallas_call(
        _stage_a_body,
        grid=(GRID_MM,),
        in_specs=[
            pl.BlockSpec((BLK, C), row),
            pl.BlockSpec((BLK, C), row),
            pl.BlockSpec((C, C), rep),
            pl.BlockSpec((1, C), rep),
            pl.BlockSpec((C, 2 * M * K), rep),
            pl.BlockSpec((1, 2 * M * K), rep),
            pl.BlockSpec((C, M * K), rep),
            pl.BlockSpec((1, M * K), rep),
            pl.BlockSpec((C, C), rep),
            pl.BlockSpec((1, C), rep),
        ],
        out_specs=[
            pl.BlockSpec((BLK, 2 * M * K), row),
            pl.BlockSpec((BLK, M * K), row),
            pl.BlockSpec((BLK, C), row),
        ],
        out_shape=[
            jax.ShapeDtypeStruct((N, 2 * M * K), jnp.float32),
            jax.ShapeDtypeStruct((N, M * K), jnp.float32),
            jax.ShapeDtypeStruct((N, C), jnp.float32),
        ],
        interpret=_INTERPRET,
    )(q2, f2, wzT, bz2, woffT, boff2, wattT, batt2, wpT, bp2)


def _stage_b_body(offx_ref, offy_ref, phix_ref, phiy_ref,
                  i00, i10, i01, i11, c00, c10, c01, c11):
    ix = (phix_ref[...] + offx_ref[...]) * (W / (W - 1.0)) - 0.5
    iy = (phiy_ref[...] + offy_ref[...]) * (H / (H - 1.0)) - 0.5
    x0 = jnp.floor(ix)
    y0 = jnp.floor(iy)
    wx1 = ix - x0
    wy1 = iy - y0
    wx0 = 1.0 - wx1
    wy0 = 1.0 - wy1
    x1 = x0 + 1.0
    y1 = y0 + 1.0

    def tap(xf, yf, wgt, iref, cref):
        valid = ((xf >= 0) & (xf <= W - 1) & (yf >= 0) & (yf <= H - 1))
        xc = jnp.clip(xf, 0.0, W - 1.0)
        yc = jnp.clip(yf, 0.0, H - 1.0)
        iref[...] = (yc * W + xc).astype(jnp.int32)
        cref[...] = wgt * valid.astype(jnp.float32)

    tap(x0, y0, wx0 * wy0, i00, c00)
    tap(x1, y0, wx1 * wy0, i10, c10)
    tap(x0, y1, wx0 * wy1, i01, c01)
    tap(x1, y1, wx1 * wy1, i11, c11)


def _stage_b(offx, offy, phix, phiy):
    TB = 4
    blk = lambda i: (i, 0, 0)
    out_spec = pl.BlockSpec((TB, K, HW), blk)
    return pl.pallas_call(
        _stage_b_body,
        grid=(T // TB,),
        in_specs=[
            pl.BlockSpec((TB, K, HW), blk),
            pl.BlockSpec((TB, K, HW), blk),
            pl.BlockSpec((TB, 1, HW), blk),
            pl.BlockSpec((TB, 1, HW), blk),
        ],
        out_specs=[out_spec] * 8,
        out_shape=[jax.ShapeDtypeStruct((T, K, HW), jnp.int32)] * 4
        + [jax.ShapeDtypeStruct((T, K, HW), jnp.float32)] * 4,
        interpret=_INTERPRET,
    )(offx, offy, phix, phiy)


def _sc_sample(tables, idxs, cfs):
    mesh = plsc.VectorSubcoreMesh(core_axis_name="c", subcore_axis_name="s")

    @functools.partial(
        pl.kernel,
        out_type=jax.ShapeDtypeStruct((T, NPOS * CV // 2), jnp.float32),
        mesh=mesh,
        scratch_types=[
            pltpu.VMEM((HW * CV,), jnp.float32),
            [pltpu.VMEM((CHUNKP,), jnp.int32) for _ in range(4)],
            [pltpu.VMEM((CHUNKP,), jnp.float32) for _ in range(4)],
            pltpu.VMEM((CHUNKP * CV // 2,), jnp.float32),
        ],
    )
    def samp(tab_hbm, i0, i1, i2, i3, c0, c1, c2, c3, out_hbm,
             tab_v, idx_vs, cf_vs, out_v):
        t = lax.axis_index("s") * 2 + lax.axis_index("c")
        pltpu.sync_copy(tab_hbm.at[t], tab_v)
        idx_hs = [i0, i1, i2, i3]
        cf_hs = [c0, c1, c2, c3]

        def to_bf_bits(acc):
            xi = lax.bitcast_convert_type(acc, jnp.int32)
            return xi + 0x7FFF + ((xi >> 16) & 1)

        def chunk_body(ci, carry):
            base = ci * CHUNKP
            for j in range(4):
                pltpu.sync_copy(idx_hs[j].at[t, pl.ds(base, CHUNKP)], idx_vs[j])
                pltpu.sync_copy(cf_hs[j].at[t, pl.ds(base, CHUNKP)], cf_vs[j])

            def g_body(g, carry2):
                ivs = [idx_vs[j][pl.ds(g * 16, 16)] for j in range(4)]
                wvs = [cf_vs[j][pl.ds(g * 16, 16)] for j in range(4)]
                for p in range(16):
                    accs = [jnp.zeros((16,), jnp.float32)
                            for _ in range(CV // 16)]
                    for j in range(4):
                        lin = ivs[j][p]
                        wgt = wvs[j][p]
                        for c in range(CV // 16):
                            accs[c] = accs[c] + wgt * tab_v[
                                pl.ds(lin * CV + c * 16, 16)]
                    for b in range(CV // 32):
                        lo = to_bf_bits(accs[2 * b])
                        hi = to_bf_bits(accs[2 * b + 1])
                        word = ((lo >> 16) & 0xFFFF) | (
                            hi & jnp.int32(-65536))
                        out_v[pl.ds((g * 16 + p) * (CV // 2) + b * 16,
                                    16)] = lax.bitcast_convert_type(
                                        word, jnp.float32)
                return carry2

            lax.fori_loop(0, CHUNKP // 16, g_body, 0)
            pltpu.sync_copy(out_v, out_hbm.at[
                t, pl.ds(base * (CV // 2), CHUNKP * CV // 2)])
            return carry

        lax.fori_loop(0, NCHUNKP, chunk_body, 0)

    return samp(tables, idxs[0], idxs[1], idxs[2], idxs[3],
                cfs[0], cfs[1], cfs[2], cfs[3])


def _jnp_sample(tables, idxs, cfs):
    sig = jnp.asarray(_SIGMA, dtype=jnp.int32)
    tab3 = tables.reshape(T, HW, CV)
    idx4 = jnp.stack(idxs, axis=2)
    cf4 = jnp.stack(cfs, axis=2)
    rows = jax.vmap(lambda tab, ii: tab[ii])(tab3, idx4)
    samp = jnp.einsum('tpjc,tpj->tpc', rows, cf4)
    samp_bf = samp[..., sig].astype(jnp.bfloat16).reshape(T, NPOS * CV // 2, 2)
    return lax.bitcast_convert_type(samp_bf, jnp.float32)


def _stage_d_body(s_ref, a_ref, hte_ref, hto_ref, ge_ref, go_ref, o_ref):
    a = a_ref[...]
    amax = jnp.max(a, axis=1, keepdims=True)
    e = jnp.exp(a - amax)
    attn = e / jnp.sum(e, axis=1, keepdims=True)
    wi = lax.bitcast_convert_type(s_ref[...], jnp.int32)
    se = lax.bitcast_convert_type(wi << 16, jnp.float32)
    so = lax.bitcast_convert_type(wi & jnp.int32(-65536), jnp.float32)
    ae = jnp.dot(attn, hte_ref[...], preferred_element_type=jnp.float32)
    ao = jnp.dot(attn, hto_ref[...], preferred_element_type=jnp.float32)
    o_ref[...] = (
        jnp.dot(se * ae, ge_ref[...], preferred_element_type=jnp.float32)
        + jnp.dot(so * ao, go_ref[...], preferred_element_type=jnp.float32))


def _stage_d(stacked2w, attn2, hte, hto, ge, go):
    row = lambda i: (i, 0)
    rep = lambda i: (0, 0)
    return pl.pallas_call(
        _stage_d_body,
        grid=(T,),
        in_specs=[
            pl.BlockSpec((HW, C // 2), row),
            pl.BlockSpec((HW, K), row),
            pl.BlockSpec((K, C // 2), rep),
            pl.BlockSpec((K, C // 2), rep),
            pl.BlockSpec((C // 2, CV), rep),
            pl.BlockSpec((C // 2, CV), rep),
        ],
        out_specs=pl.BlockSpec((HW, CV), row),
        out_shape=jax.ShapeDtypeStruct((T * HW, CV), jnp.float32),
        interpret=_INTERPRET,
    )(stacked2w, attn2, hte, hto, ge, go)


def _stage_e_body(x_ref, w_ref, b_ref, o_ref):
    o_ref[...] = jnp.dot(x_ref[...], w_ref[...],
                         preferred_element_type=jnp.float32) + b_ref[...]


def _stage_e(x2, wmT, bm2):
    row = lambda i: (i, 0)
    rep = lambda i: (0, 0)
    return pl.pallas_call(
        _stage_e_body,
        grid=(GRID_MM,),
        in_specs=[
            pl.BlockSpec((BLK, C), row),
            pl.BlockSpec((C, C), rep),
            pl.BlockSpec((1, C), rep),
        ],
        out_specs=pl.BlockSpec((BLK, C), row),
        out_shape=jax.ShapeDtypeStruct((N, C), jnp.float32),
        interpret=_INTERPRET,
    )(x2, wmT, bm2)


def kernel(q, features, ref, Wz, bz, Woff, boff, Watt, batt, Wp, bp, Wm, bm):
    q2 = q.reshape(N, C)
    f2 = features[0].reshape(N, C)

    off_raw, att_raw, wp2 = _stage_a(
        q2, f2,
        Wz.T, bz.reshape(1, C),
        Woff.T, boff.reshape(1, 2 * M * K),
        Watt.T, batt.reshape(1, M * K),
        Wp.T, bp.reshape(1, C))

    off6 = off_raw.reshape(BS, H, W, M, K, 2)
    offx = jnp.transpose(off6[..., 0], (0, 3, 4, 2, 1)).reshape(T, K, HW)
    offy = jnp.transpose(off6[..., 1], (0, 3, 4, 2, 1)).reshape(T, K, HW)
    phix = jnp.tile(jnp.transpose(ref[..., 0], (0, 2, 1)).reshape(BS, 1, HW)
                    * (W - 1.0), (M, 1, 1))
    phiy = jnp.tile(jnp.transpose(ref[..., 1], (0, 2, 1)).reshape(BS, 1, HW)
                    * (H - 1.0), (M, 1, 1))

    i00, i10, i01, i11, c00, c10, c01, c11 = _stage_b(offx, offy, phix, phiy)

    idxs = [a.reshape(T, NPOS) for a in (i00, i10, i01, i11)]
    cfs = [a.reshape(T, NPOS) for a in (c00, c10, c01, c11)]

    sig = jnp.asarray(_SIGMA, dtype=jnp.int32)
    tables = wp2.reshape(BS, HW, M, CV).transpose(0, 2, 1, 3).reshape(
        T, HW * CV)

    if _USE_SC:
        samp = _sc_sample(tables, idxs, cfs)
    else:
        samp = _jnp_sample(tables, idxs, cfs)

    stacked2w = samp.reshape(T * HW, C // 2)
    attn2 = att_raw.reshape(BS, HW, M, K).transpose(0, 2, 1, 3).reshape(T * HW, K)

    ii = jnp.arange(C, dtype=jnp.int32)
    e_sem = (ii // CV) * CV + sig[ii % CV]
    htile = (e_sem[None, :] % K == jnp.arange(K, dtype=jnp.int32)[:, None]
             ).astype(jnp.float32)
    gsum = (e_sem[:, None] // K == jnp.arange(CV, dtype=jnp.int32)[None, :]
            ).astype(jnp.float32)

    att_out2 = _stage_d(stacked2w, attn2, htile[:, 0::2], htile[:, 1::2],
                        gsum[0::2, :], gsum[1::2, :])

    att_out = att_out2.reshape(BS, M, HW, CV).transpose(0, 2, 1, 3).reshape(N, C)
    final = _stage_e(att_out, Wm.T, bm.reshape(1, C))
    return final.reshape(BS, H, W, C)


# speedup vs baseline: 18.8394x; 1.0409x over previous
import functools

import jax
import jax.numpy as jnp
from jax import lax
from jax.experimental import pallas as pl
from jax.experimental.pallas import tpu as pltpu
from jax.experimental.pallas import tpu_sc as plsc

_INTERPRET = False
_USE_SC = True

C = 768
M = 8
K = 8
CV = C // M
H = 32
W = 32
BS = 4
HW = H * W
N = BS * HW
T = BS * M
NPOS = K * HW

BLK = 512
GRID_MM = N // BLK

CHUNKP = 256
NCHUNKP = NPOS // CHUNKP

_SIGMA = [32 * (p // 32) + (p % 2) * 16 + (p % 32) // 2 for p in range(CV)]


def _stage_a_body(q_ref, f_ref, wz_ref, bz_ref, woff_ref, boff_ref,
                  watt_ref, batt_ref, wp_ref, bp_ref,
                  off_ref, att_ref, wpo_ref):
    zq = jnp.dot(q_ref[...], wz_ref[...],
                 preferred_element_type=jnp.float32) + bz_ref[...]
    off_ref[...] = jnp.dot(zq, woff_ref[...],
                           preferred_element_type=jnp.float32) + boff_ref[...]
    att_ref[...] = jnp.dot(zq, watt_ref[...],
                           preferred_element_type=jnp.float32) + batt_ref[...]
    wpo_ref[...] = jnp.dot(f_ref[...], wp_ref[...],
                           preferred_element_type=jnp.float32) + bp_ref[...]


def _stage_a(q2, f2, wzT, bz2, woffT, boff2, wattT, batt2, wpT, bp2):
    row = lambda i: (i, 0)
    rep = lambda i: (0, 0)
    return pl.pallas_call(
        _stage_a_body,
        grid=(GRID_MM,),
        in_specs=[
            pl.BlockSpec((BLK, C), row),
            pl.BlockSpec((BLK, C), row),
            pl.BlockSpec((C, C), rep),
            pl.BlockSpec((1, C), rep),
            pl.BlockSpec((C, 2 * M * K), rep),
            pl.BlockSpec((1, 2 * M * K), rep),
            pl.BlockSpec((C, M * K), rep),
            pl.BlockSpec((1, M * K), rep),
            pl.BlockSpec((C, C), rep),
            pl.BlockSpec((1, C), rep),
        ],
        out_specs=[
            pl.BlockSpec((BLK, 2 * M * K), row),
            pl.BlockSpec((BLK, M * K), row),
            pl.BlockSpec((BLK, C), row),
        ],
        out_shape=[
            jax.ShapeDtypeStruct((N, 2 * M * K), jnp.float32),
            jax.ShapeDtypeStruct((N, M * K), jnp.float32),
            jax.ShapeDtypeStruct((N, C), jnp.float32),
        ],
        interpret=_INTERPRET,
    )(q2, f2, wzT, bz2, woffT, boff2, wattT, batt2, wpT, bp2)


def _stage_b_body(offx_ref, offy_ref, phix_ref, phiy_ref,
                  i00, i10, i01, i11, c00, c10, c01, c11):
    ix = (phix_ref[...] + offx_ref[...]) * (W / (W - 1.0)) - 0.5
    iy = (phiy_ref[...] + offy_ref[...]) * (H / (H - 1.0)) - 0.5
    x0 = jnp.floor(ix)
    y0 = jnp.floor(iy)
    wx1 = ix - x0
    wy1 = iy - y0
    wx0 = 1.0 - wx1
    wy0 = 1.0 - wy1
    x1 = x0 + 1.0
    y1 = y0 + 1.0

    def tap(xf, yf, wgt, iref, cref):
        valid = ((xf >= 0) & (xf <= W - 1) & (yf >= 0) & (yf <= H - 1))
        xc = jnp.clip(xf, 0.0, W - 1.0)
        yc = jnp.clip(yf, 0.0, H - 1.0)
        iref[...] = (yc * W + xc).astype(jnp.int32)
        cref[...] = wgt * valid.astype(jnp.float32)

    tap(x0, y0, wx0 * wy0, i00, c00)
    tap(x1, y0, wx1 * wy0, i10, c10)
    tap(x0, y1, wx0 * wy1, i01, c01)
    tap(x1, y1, wx1 * wy1, i11, c11)


def _stage_b(offx, offy, phix, phiy):
    TB = 4
    blk = lambda i: (i, 0, 0)
    out_spec = pl.BlockSpec((TB, K, HW), blk)
    return pl.pallas_call(
        _stage_b_body,
        grid=(T // TB,),
        in_specs=[
            pl.BlockSpec((TB, K, HW), blk),
            pl.BlockSpec((TB, K, HW), blk),
            pl.BlockSpec((TB, 1, HW), blk),
            pl.BlockSpec((TB, 1, HW), blk),
        ],
        out_specs=[out_spec] * 8,
        out_shape=[jax.ShapeDtypeStruct((T, K, HW), jnp.int32)] * 4
        + [jax.ShapeDtypeStruct((T, K, HW), jnp.float32)] * 4,
        interpret=_INTERPRET,
    )(offx, offy, phix, phiy)


def _sc_sample(tables, idxs, cfs):
    mesh = plsc.VectorSubcoreMesh(core_axis_name="c", subcore_axis_name="s")

    @functools.partial(
        pl.kernel,
        out_type=jax.ShapeDtypeStruct((T, NPOS * CV // 2), jnp.float32),
        mesh=mesh,
        scratch_types=[
            pltpu.VMEM((HW * CV,), jnp.float32),
            [pltpu.VMEM((CHUNKP,), jnp.int32) for _ in range(4)],
            [pltpu.VMEM((CHUNKP,), jnp.float32) for _ in range(4)],
            pltpu.VMEM((CHUNKP * CV // 2,), jnp.float32),
        ],
    )
    def samp(tab_hbm, i0, i1, i2, i3, c0, c1, c2, c3, out_hbm,
             tab_v, idx_vs, cf_vs, out_v):
        t = lax.axis_index("s") * 2 + lax.axis_index("c")
        pltpu.sync_copy(tab_hbm.at[t], tab_v)
        idx_hs = [i0, i1, i2, i3]
        cf_hs = [c0, c1, c2, c3]

        def to_bf_bits(acc):
            xi = lax.bitcast_convert_type(acc, jnp.int32)
            return xi + 0x8000

        def chunk_body(ci, carry):
            base = ci * CHUNKP
            for j in range(4):
                pltpu.sync_copy(idx_hs[j].at[t, pl.ds(base, CHUNKP)], idx_vs[j])
                pltpu.sync_copy(cf_hs[j].at[t, pl.ds(base, CHUNKP)], cf_vs[j])

            def g_body(g, carry2):
                ivs = [idx_vs[j][pl.ds(g * 16, 16)] for j in range(4)]
                wvs = [cf_vs[j][pl.ds(g * 16, 16)] for j in range(4)]
                for p in range(16):
                    accs = [jnp.zeros((16,), jnp.float32)
                            for _ in range(CV // 16)]
                    for j in range(4):
                        lin = ivs[j][p]
                        wgt = wvs[j][p]
                        for c in range(CV // 16):
                            accs[c] = accs[c] + wgt * tab_v[
                                pl.ds(lin * CV + c * 16, 16)]
                    for b in range(CV // 32):
                        lo = to_bf_bits(accs[2 * b])
                        hi = to_bf_bits(accs[2 * b + 1])
                        word = ((lo >> 16) & 0xFFFF) | (
                            hi & jnp.int32(-65536))
                        out_v[pl.ds((g * 16 + p) * (CV // 2) + b * 16,
                                    16)] = lax.bitcast_convert_type(
                                        word, jnp.float32)
                return carry2

            lax.fori_loop(0, CHUNKP // 16, g_body, 0)
            pltpu.sync_copy(out_v, out_hbm.at[
                t, pl.ds(base * (CV // 2), CHUNKP * CV // 2)])
            return carry

        lax.fori_loop(0, NCHUNKP, chunk_body, 0)

    return samp(tables, idxs[0], idxs[1], idxs[2], idxs[3],
                cfs[0], cfs[1], cfs[2], cfs[3])


def _jnp_sample(tables, idxs, cfs):
    sig = jnp.asarray(_SIGMA, dtype=jnp.int32)
    tab3 = tables.reshape(T, HW, CV)
    idx4 = jnp.stack(idxs, axis=2)
    cf4 = jnp.stack(cfs, axis=2)
    rows = jax.vmap(lambda tab, ii: tab[ii])(tab3, idx4)
    samp = jnp.einsum('tpjc,tpj->tpc', rows, cf4)
    samp_bf = samp[..., sig].astype(jnp.bfloat16).reshape(T, NPOS * CV // 2, 2)
    return lax.bitcast_convert_type(samp_bf, jnp.float32)


def _stage_d_body(s_ref, a_ref, hte_ref, hto_ref, ge_ref, go_ref, o_ref):
    a = a_ref[...]
    amax = jnp.max(a, axis=1, keepdims=True)
    e = jnp.exp(a - amax)
    attn = e / jnp.sum(e, axis=1, keepdims=True)
    wi = lax.bitcast_convert_type(s_ref[...], jnp.int32)
    se = lax.bitcast_convert_type(wi << 16, jnp.float32)
    so = lax.bitcast_convert_type(wi & jnp.int32(-65536), jnp.float32)
    ae = jnp.dot(attn, hte_ref[...], preferred_element_type=jnp.float32)
    ao = jnp.dot(attn, hto_ref[...], preferred_element_type=jnp.float32)
    o_ref[...] = (
        jnp.dot(se * ae, ge_ref[...], preferred_element_type=jnp.float32)
        + jnp.dot(so * ao, go_ref[...], preferred_element_type=jnp.float32))


def _stage_d(stacked2w, attn2, hte, hto, ge, go):
    row = lambda i: (i, 0)
    rep = lambda i: (0, 0)
    return pl.pallas_call(
        _stage_d_body,
        grid=(T,),
        in_specs=[
            pl.BlockSpec((HW, C // 2), row),
            pl.BlockSpec((HW, K), row),
            pl.BlockSpec((K, C // 2), rep),
            pl.BlockSpec((K, C // 2), rep),
            pl.BlockSpec((C // 2, CV), rep),
            pl.BlockSpec((C // 2, CV), rep),
        ],
        out_specs=pl.BlockSpec((HW, CV), row),
        out_shape=jax.ShapeDtypeStruct((T * HW, CV), jnp.float32),
        interpret=_INTERPRET,
    )(stacked2w, attn2, hte, hto, ge, go)


def _stage_e_body(x_ref, w_ref, b_ref, o_ref):
    o_ref[...] = jnp.dot(x_ref[...], w_ref[...],
                         preferred_element_type=jnp.float32) + b_ref[...]


def _stage_e(x2, wmT, bm2):
    row = lambda i: (i, 0)
    rep = lambda i: (0, 0)
    return pl.pallas_call(
        _stage_e_body,
        grid=(GRID_MM,),
        in_specs=[
            pl.BlockSpec((BLK, C), row),
            pl.BlockSpec((C, C), rep),
            pl.BlockSpec((1, C), rep),
        ],
        out_specs=pl.BlockSpec((BLK, C), row),
        out_shape=jax.ShapeDtypeStruct((N, C), jnp.float32),
        interpret=_INTERPRET,
    )(x2, wmT, bm2)


def kernel(q, features, ref, Wz, bz, Woff, boff, Watt, batt, Wp, bp, Wm, bm):
    q2 = q.reshape(N, C)
    f2 = features[0].reshape(N, C)

    off_raw, att_raw, wp2 = _stage_a(
        q2, f2,
        Wz.T, bz.reshape(1, C),
        Woff.T, boff.reshape(1, 2 * M * K),
        Watt.T, batt.reshape(1, M * K),
        Wp.T, bp.reshape(1, C))

    off6 = off_raw.reshape(BS, H, W, M, K, 2)
    offx = jnp.transpose(off6[..., 0], (0, 3, 4, 2, 1)).reshape(T, K, HW)
    offy = jnp.transpose(off6[..., 1], (0, 3, 4, 2, 1)).reshape(T, K, HW)
    phix = jnp.tile(jnp.transpose(ref[..., 0], (0, 2, 1)).reshape(BS, 1, HW)
                    * (W - 1.0), (M, 1, 1))
    phiy = jnp.tile(jnp.transpose(ref[..., 1], (0, 2, 1)).reshape(BS, 1, HW)
                    * (H - 1.0), (M, 1, 1))

    i00, i10, i01, i11, c00, c10, c01, c11 = _stage_b(offx, offy, phix, phiy)

    idxs = [a.reshape(T, NPOS) for a in (i00, i10, i01, i11)]
    cfs = [a.reshape(T, NPOS) for a in (c00, c10, c01, c11)]

    sig = jnp.asarray(_SIGMA, dtype=jnp.int32)
    tables = wp2.reshape(BS, HW, M, CV).transpose(0, 2, 1, 3).reshape(
        T, HW * CV)

    if _USE_SC:
        samp = _sc_sample(tables, idxs, cfs)
    else:
        samp = _jnp_sample(tables, idxs, cfs)

    stacked2w = samp.reshape(T * HW, C // 2)
    attn2 = att_raw.reshape(BS, HW, M, K).transpose(0, 2, 1, 3).reshape(T * HW, K)

    ii = jnp.arange(C, dtype=jnp.int32)
    e_sem = (ii // CV) * CV + sig[ii % CV]
    htile = (e_sem[None, :] % K == jnp.arange(K, dtype=jnp.int32)[:, None]
             ).astype(jnp.float32)
    gsum = (e_sem[:, None] // K == jnp.arange(CV, dtype=jnp.int32)[None, :]
            ).astype(jnp.float32)

    att_out2 = _stage_d(stacked2w, attn2, htile[:, 0::2], htile[:, 1::2],
                        gsum[0::2, :], gsum[1::2, :])

    att_out = att_out2.reshape(BS, M, HW, CV).transpose(0, 2, 1, 3).reshape(N, C)
    final = _stage_e(att_out, Wm.T, bm.reshape(1, C))
    return final.reshape(BS, H, W, C)


# speedup vs baseline: 18.8517x; 1.0007x over previous
import functools

import jax
import jax.numpy as jnp
from jax import lax
from jax.experimental import pallas as pl
from jax.experimental.pallas import tpu as pltpu
from jax.experimental.pallas import tpu_sc as plsc

C = 768
M = 8
K = 8
CV = C // M
H = 32
W = 32
BS = 4
HW = H * W
N = BS * HW
T = BS * M
NPOS = K * HW

BLK = 512
GRID_MM = N // BLK

CHUNKP = 256
NCHUNKP = NPOS // CHUNKP

_SIGMA = [32 * (p // 32) + (p % 2) * 16 + (p % 32) // 2 for p in range(CV)]


def _stage_a_body(q_ref, f_ref, wz_ref, bz_ref, woff_ref, boff_ref,
                  watt_ref, batt_ref, wp_ref, bp_ref,
                  off_ref, att_ref, wpo_ref):
    zq = jnp.dot(q_ref[...], wz_ref[...],
                 preferred_element_type=jnp.float32) + bz_ref[...]
    off_ref[...] = jnp.dot(zq, woff_ref[...],
                           preferred_element_type=jnp.float32) + boff_ref[...]
    att_ref[...] = jnp.dot(zq, watt_ref[...],
                           preferred_element_type=jnp.float32) + batt_ref[...]
    wpo_ref[...] = jnp.dot(f_ref[...], wp_ref[...],
                           preferred_element_type=jnp.float32) + bp_ref[...]


def _stage_a(q2, f2, wzT, bz2, woffT, boff2, wattT, batt2, wpT, bp2):
    row = lambda i: (i, 0)
    rep = lambda i: (0, 0)
    return pl.pallas_call(
        _stage_a_body,
        grid=(GRID_MM,),
        in_specs=[
            pl.BlockSpec((BLK, C), row),
            pl.BlockSpec((BLK, C), row),
            pl.BlockSpec((C, C), rep),
            pl.BlockSpec((1, C), rep),
            pl.BlockSpec((C, 2 * M * K), rep),
            pl.BlockSpec((1, 2 * M * K), rep),
            pl.BlockSpec((C, M * K), rep),
            pl.BlockSpec((1, M * K), rep),
            pl.BlockSpec((C, C), rep),
            pl.BlockSpec((1, C), rep),
        ],
        out_specs=[
            pl.BlockSpec((BLK, 2 * M * K), row),
            pl.BlockSpec((BLK, M * K), row),
            pl.BlockSpec((BLK, C), row),
        ],
        out_shape=[
            jax.ShapeDtypeStruct((N, 2 * M * K), jnp.float32),
            jax.ShapeDtypeStruct((N, M * K), jnp.float32),
            jax.ShapeDtypeStruct((N, C), jnp.float32),
        ],
    )(q2, f2, wzT, bz2, woffT, boff2, wattT, batt2, wpT, bp2)


def _stage_b_body(offx_ref, offy_ref, phix_ref, phiy_ref,
                  i00, i10, i01, i11, c00, c10, c01, c11):
    ix = (phix_ref[...] + offx_ref[...]) * (W / (W - 1.0)) - 0.5
    iy = (phiy_ref[...] + offy_ref[...]) * (H / (H - 1.0)) - 0.5
    x0 = jnp.floor(ix)
    y0 = jnp.floor(iy)
    wx1 = ix - x0
    wy1 = iy - y0
    wx0 = 1.0 - wx1
    wy0 = 1.0 - wy1
    x1 = x0 + 1.0
    y1 = y0 + 1.0

    def tap(xf, yf, wgt, iref, cref):
        valid = ((xf >= 0) & (xf <= W - 1) & (yf >= 0) & (yf <= H - 1))
        xc = jnp.clip(xf, 0.0, W - 1.0)
        yc = jnp.clip(yf, 0.0, H - 1.0)
        iref[...] = (yc * W + xc).astype(jnp.int32)
        cref[...] = wgt * valid.astype(jnp.float32)

    tap(x0, y0, wx0 * wy0, i00, c00)
    tap(x1, y0, wx1 * wy0, i10, c10)
    tap(x0, y1, wx0 * wy1, i01, c01)
    tap(x1, y1, wx1 * wy1, i11, c11)


def _stage_b(offx, offy, phix, phiy):
    TB = 4
    blk = lambda i: (i, 0, 0)
    out_spec = pl.BlockSpec((TB, K, HW), blk)
    return pl.pallas_call(
        _stage_b_body,
        grid=(T // TB,),
        in_specs=[
            pl.BlockSpec((TB, K, HW), blk),
            pl.BlockSpec((TB, K, HW), blk),
            pl.BlockSpec((TB, 1, HW), blk),
            pl.BlockSpec((TB, 1, HW), blk),
        ],
        out_specs=[out_spec] * 8,
        out_shape=[jax.ShapeDtypeStruct((T, K, HW), jnp.int32)] * 4
        + [jax.ShapeDtypeStruct((T, K, HW), jnp.float32)] * 4,
    )(offx, offy, phix, phiy)


def _sc_sample(tables, idxs, cfs):
    mesh = plsc.VectorSubcoreMesh(core_axis_name="c", subcore_axis_name="s")

    @functools.partial(
        pl.kernel,
        out_type=jax.ShapeDtypeStruct((T, NPOS * CV // 2), jnp.float32),
        mesh=mesh,
        scratch_types=[
            pltpu.VMEM((HW * CV,), jnp.float32),
            [pltpu.VMEM((CHUNKP,), jnp.int32) for _ in range(4)],
            [pltpu.VMEM((CHUNKP,), jnp.float32) for _ in range(4)],
            pltpu.VMEM((CHUNKP * CV // 2,), jnp.float32),
        ],
    )
    def samp(tab_hbm, i0, i1, i2, i3, c0, c1, c2, c3, out_hbm,
             tab_v, idx_vs, cf_vs, out_v):
        t = lax.axis_index("s") * 2 + lax.axis_index("c")
        pltpu.sync_copy(tab_hbm.at[t], tab_v)
        idx_hs = [i0, i1, i2, i3]
        cf_hs = [c0, c1, c2, c3]

        def to_bf_bits(acc):
            xi = lax.bitcast_convert_type(acc, jnp.int32)
            return xi + 0x8000

        def chunk_body(ci, carry):
            base = ci * CHUNKP
            for j in range(4):
                pltpu.sync_copy(idx_hs[j].at[t, pl.ds(base, CHUNKP)], idx_vs[j])
                pltpu.sync_copy(cf_hs[j].at[t, pl.ds(base, CHUNKP)], cf_vs[j])

            def g_body(g, carry2):
                ivs = [idx_vs[j][pl.ds(g * 16, 16)] for j in range(4)]
                wvs = [cf_vs[j][pl.ds(g * 16, 16)] for j in range(4)]
                for p in range(16):
                    accs = [jnp.zeros((16,), jnp.float32)
                            for _ in range(CV // 16)]
                    for j in range(4):
                        lin = ivs[j][p]
                        wgt = wvs[j][p]
                        for c in range(CV // 16):
                            accs[c] = accs[c] + wgt * tab_v[
                                pl.ds(lin * CV + c * 16, 16)]
                    for b in range(CV // 32):
                        lo = to_bf_bits(accs[2 * b])
                        hi = to_bf_bits(accs[2 * b + 1])
                        word = ((lo >> 16) & 0xFFFF) | (
                            hi & jnp.int32(-65536))
                        out_v[pl.ds((g * 16 + p) * (CV // 2) + b * 16,
                                    16)] = lax.bitcast_convert_type(
                                        word, jnp.float32)
                return carry2

            lax.fori_loop(0, CHUNKP // 16, g_body, 0)
            pltpu.sync_copy(out_v, out_hbm.at[
                t, pl.ds(base * (CV // 2), CHUNKP * CV // 2)])
            return carry

        lax.fori_loop(0, NCHUNKP, chunk_body, 0)

    return samp(tables, idxs[0], idxs[1], idxs[2], idxs[3],
                cfs[0], cfs[1], cfs[2], cfs[3])


def _stage_d_body(s_ref, a_ref, hte_ref, hto_ref, ge_ref, go_ref, o_ref):
    a = a_ref[...]
    amax = jnp.max(a, axis=1, keepdims=True)
    e = jnp.exp(a - amax)
    attn = e / jnp.sum(e, axis=1, keepdims=True)
    wi = lax.bitcast_convert_type(s_ref[...], jnp.int32)
    se = lax.bitcast_convert_type(wi << 16, jnp.float32)
    so = lax.bitcast_convert_type(wi & jnp.int32(-65536), jnp.float32)
    ae = jnp.dot(attn, hte_ref[...], preferred_element_type=jnp.float32)
    ao = jnp.dot(attn, hto_ref[...], preferred_element_type=jnp.float32)
    o_ref[...] = (
        jnp.dot(se * ae, ge_ref[...], preferred_element_type=jnp.float32)
        + jnp.dot(so * ao, go_ref[...], preferred_element_type=jnp.float32))


def _stage_d(stacked2w, attn2, hte, hto, ge, go):
    row = lambda i: (i, 0)
    rep = lambda i: (0, 0)
    return pl.pallas_call(
        _stage_d_body,
        grid=(T,),
        in_specs=[
            pl.BlockSpec((HW, C // 2), row),
            pl.BlockSpec((HW, K), row),
            pl.BlockSpec((K, C // 2), rep),
            pl.BlockSpec((K, C // 2), rep),
            pl.BlockSpec((C // 2, CV), rep),
            pl.BlockSpec((C // 2, CV), rep),
        ],
        out_specs=pl.BlockSpec((HW, CV), row),
        out_shape=jax.ShapeDtypeStruct((T * HW, CV), jnp.float32),
    )(stacked2w, attn2, hte, hto, ge, go)


def _stage_e_body(x_ref, w_ref, b_ref, o_ref):
    o_ref[...] = jnp.dot(x_ref[...], w_ref[...],
                         preferred_element_type=jnp.float32) + b_ref[...]


def _stage_e(x2, wmT, bm2):
    row = lambda i: (i, 0)
    rep = lambda i: (0, 0)
    return pl.pallas_call(
        _stage_e_body,
        grid=(GRID_MM,),
        in_specs=[
            pl.BlockSpec((BLK, C), row),
            pl.BlockSpec((C, C), rep),
            pl.BlockSpec((1, C), rep),
        ],
        out_specs=pl.BlockSpec((BLK, C), row),
        out_shape=jax.ShapeDtypeStruct((N, C), jnp.float32),
    )(x2, wmT, bm2)


def kernel(q, features, ref, Wz, bz, Woff, boff, Watt, batt, Wp, bp, Wm, bm):
    q2 = q.reshape(N, C)
    f2 = features[0].reshape(N, C)

    off_raw, att_raw, wp2 = _stage_a(
        q2, f2,
        Wz.T, bz.reshape(1, C),
        Woff.T, boff.reshape(1, 2 * M * K),
        Watt.T, batt.reshape(1, M * K),
        Wp.T, bp.reshape(1, C))

    off6 = off_raw.reshape(BS, H, W, M, K, 2)
    offx = jnp.transpose(off6[..., 0], (0, 3, 4, 2, 1)).reshape(T, K, HW)
    offy = jnp.transpose(off6[..., 1], (0, 3, 4, 2, 1)).reshape(T, K, HW)
    phix = jnp.tile(jnp.transpose(ref[..., 0], (0, 2, 1)).reshape(BS, 1, HW)
                    * (W - 1.0), (M, 1, 1))
    phiy = jnp.tile(jnp.transpose(ref[..., 1], (0, 2, 1)).reshape(BS, 1, HW)
                    * (H - 1.0), (M, 1, 1))

    i00, i10, i01, i11, c00, c10, c01, c11 = _stage_b(offx, offy, phix, phiy)

    idxs = [a.reshape(T, NPOS) for a in (i00, i10, i01, i11)]
    cfs = [a.reshape(T, NPOS) for a in (c00, c10, c01, c11)]

    sig = jnp.asarray(_SIGMA, dtype=jnp.int32)
    tables = wp2.reshape(BS, HW, M, CV).transpose(0, 2, 1, 3).reshape(
        T, HW * CV)

    samp = _sc_sample(tables, idxs, cfs)

    stacked2w = samp.reshape(T * HW, C // 2)
    attn2 = att_raw.reshape(BS, HW, M, K).transpose(0, 2, 1, 3).reshape(T * HW, K)

    ii = jnp.arange(C, dtype=jnp.int32)
    e_sem = (ii // CV) * CV + sig[ii % CV]
    htile = (e_sem[None, :] % K == jnp.arange(K, dtype=jnp.int32)[:, None]
             ).astype(jnp.float32)
    gsum = (e_sem[:, None] // K == jnp.arange(CV, dtype=jnp.int32)[None, :]
            ).astype(jnp.float32)

    att_out2 = _stage_d(stacked2w, attn2, htile[:, 0::2], htile[:, 1::2],
                        gsum[0::2, :], gsum[1::2, :])

    att_out = att_out2.reshape(BS, M, HW, CV).transpose(0, 2, 1, 3).reshape(N, C)
    final = _stage_e(att_out, Wm.T, bm.reshape(1, C))
    return final.reshape(BS, H, W, C)
